# baseline (speedup 1.0000x reference)
import jax
import jax.numpy as jnp
from jax.experimental import pallas as pl
from jax.experimental.pallas import tpu as pltpu


def _make_body(nk):
    def _mlp_body(x_ref, w1_ref, b1_ref, w2_ref, b2_ref, out_ref, acc_ref):
        k = pl.program_id(1)
        xb = x_ref[...].astype(jnp.bfloat16)
        w1b = w1_ref[...].astype(jnp.bfloat16)
        part = jnp.dot(xb, w1b, preferred_element_type=jnp.float32)

        @pl.when(k == 0)
        def _():
            acc_ref[...] = part

        @pl.when(k != 0)
        def _():
            acc_ref[...] += part

        @pl.when(k == nk - 1)
        def _():
            hid = jnp.maximum(acc_ref[...] + b1_ref[...], 0.0)
            hid = hid.astype(jnp.bfloat16)
            w2b = w2_ref[...].astype(jnp.bfloat16)
            out = jnp.dot(hid, w2b, preferred_element_type=jnp.float32)
            out_ref[...] = out + b2_ref[...]

    return _mlp_body


@jax.jit
def kernel(x, w1, b1, w2, b2):
    B, S = x.shape
    H = w1.shape[1]
    A = w2.shape[1]

    TB = min(8192, B)
    nb = pl.cdiv(B, TB)
    nk = 2 if S % 2 == 0 else 1
    SK = S // nk

    return pl.pallas_call(
        _make_body(nk),
        out_shape=jax.ShapeDtypeStruct((B, A), x.dtype),
        grid=(nb, nk),
        in_specs=[
            pl.BlockSpec((TB, SK), lambda i, k: (i, k)),
            pl.BlockSpec((SK, H), lambda i, k: (k, 0)),
            pl.BlockSpec((1, H), lambda i, k: (0, 0)),
            pl.BlockSpec((H, A), lambda i, k: (0, 0)),
            pl.BlockSpec((1, A), lambda i, k: (0, 0)),
        ],
        out_specs=pl.BlockSpec((TB, A), lambda i, k: (i, 0)),
        scratch_shapes=[pltpu.VMEM((TB, H), jnp.float32)],
        compiler_params=pltpu.CompilerParams(
            dimension_semantics=("parallel", "arbitrary"),
        ),
    )(x, w1, b1, w2, b2)


# speedup vs baseline: 1.0739x; 1.0739x over previous
import jax
import jax.numpy as jnp
from jax.experimental import pallas as pl
from jax.experimental.pallas import tpu as pltpu


def _mlp_body(x_ref, w1_ref, b1_ref, w2_ref, b2_ref, out_ref):
    x = x_ref[...].astype(jnp.bfloat16)
    w1 = w1_ref[...].astype(jnp.bfloat16)
    hid = jnp.dot(x, w1, preferred_element_type=jnp.float32)
    hid = jnp.maximum(hid + b1_ref[...], 0.0).astype(jnp.bfloat16)
    w2 = w2_ref[...].astype(jnp.bfloat16)
    out = jnp.dot(hid, w2, preferred_element_type=jnp.float32)
    out_ref[...] = out + b2_ref[...]


@jax.jit
def kernel(x, w1, b1, w2, b2):
    B, S = x.shape
    H = w1.shape[1]
    A = w2.shape[1]

    TB = min(5464, B)
    nb = pl.cdiv(B, TB)

    return pl.pallas_call(
        _mlp_body,
        out_shape=jax.ShapeDtypeStruct((B, A), x.dtype),
        grid=(nb,),
        in_specs=[
            pl.BlockSpec((TB, S), lambda i: (i, 0)),
            pl.BlockSpec((S, H), lambda i: (0, 0)),
            pl.BlockSpec((1, H), lambda i: (0, 0)),
            pl.BlockSpec((H, A), lambda i: (0, 0)),
            pl.BlockSpec((1, A), lambda i: (0, 0)),
        ],
        out_specs=pl.BlockSpec((TB, A), lambda i: (i, 0)),
        compiler_params=pltpu.CompilerParams(
            dimension_semantics=("parallel",),
        ),
    )(x, w1, b1, w2, b2)


# speedup vs baseline: 1.2180x; 1.1342x over previous
import jax
import jax.numpy as jnp
from jax.experimental import pallas as pl
from jax.experimental.pallas import tpu as pltpu


def _mlp_body(x_ref, w1_ref, b1_ref, w2_ref, b2_ref, out_ref):
    x = x_ref[...].astype(jnp.bfloat16)
    w1 = w1_ref[...].astype(jnp.bfloat16)
    hid = jnp.dot(x, w1, preferred_element_type=jnp.float32)
    hid = jnp.maximum(hid + b1_ref[...], 0.0).astype(jnp.bfloat16)
    w2 = w2_ref[...].astype(jnp.bfloat16)
    out = jnp.dot(hid, w2, preferred_element_type=jnp.float32)
    out_ref[...] = out + b2_ref[...]


@jax.jit
def kernel(x, w1, b1, w2, b2):
    B, S = x.shape
    H = w1.shape[1]
    A = w2.shape[1]

    TB = min(8192, B)
    nb = pl.cdiv(B, TB)

    return pl.pallas_call(
        _mlp_body,
        out_shape=jax.ShapeDtypeStruct((B, A), x.dtype),
        grid=(nb,),
        in_specs=[
            pl.BlockSpec((TB, S), lambda i: (i, 0)),
            pl.BlockSpec((S, H), lambda i: (0, 0)),
            pl.BlockSpec((1, H), lambda i: (0, 0)),
            pl.BlockSpec((H, A), lambda i: (0, 0)),
            pl.BlockSpec((1, A), lambda i: (0, 0)),
        ],
        out_specs=pl.BlockSpec((TB, A), lambda i: (i, 0)),
        compiler_params=pltpu.CompilerParams(
            dimension_semantics=("parallel",),
        ),
    )(x, w1, b1, w2, b2)
